# baseline (speedup 1.0000x reference)
import functools

import jax
import jax.numpy as jnp
from jax import lax
from jax.experimental import pallas as pl
from jax.experimental.pallas import tpu as pltpu
from jax.experimental.pallas import tpu_sc as plsc

GRID_SIZE = 16
MARGIN = 1.0
LOSS_LAMBDA = 1.0

BN = 1024
NW = 32
L = 16


def _sc_sample_body(w_hbm, kd_hbm, d2_hbm, pos_hbm,
                    wv, idxv, wgtv, rows, kdv, posv, sem,
                    *, ppw, c, h, w):
    wid = lax.axis_index("s") * 2 + lax.axis_index("c")
    base = wid * ppw
    pltpu.sync_copy(w_hbm.at[pl.ds(base * 2, ppw * 2)], wv)
    pltpu.sync_copy(kd_hbm.at[pl.ds(base, ppw)], kdv)

    lanes = lax.iota(jnp.int32, L)
    ngroups = ppw // L

    def floorf(v):
        t = v.astype(jnp.int32).astype(jnp.float32)
        return jnp.where(v < t, t - 1.0, t)

    for g in range(ngroups):
        pyg = plsc.load_gather(wv, [(g * L + lanes) * 2])
        pxg = plsc.load_gather(wv, [(g * L + lanes) * 2 + 1])
        ys = pyg / GRID_SIZE - 0.5
        xs = pxg / GRID_SIZE - 0.5
        y0 = floorf(ys)
        x0 = floorf(xs)
        y1 = y0 + 1.0
        x1 = x0 + 1.0
        wx1 = xs - x0
        wx0 = 1.0 - wx1
        wy1 = ys - y0
        wy0 = 1.0 - wy1
        for k, (yf, xf, wgt) in enumerate((
                (y0, x0, wy0 * wx0), (y0, x1, wy0 * wx1),
                (y1, x0, wy1 * wx0), (y1, x1, wy1 * wx1))):
            valid = ((yf >= 0.0) & (yf <= h - 1.0)
                     & (xf >= 0.0) & (xf <= w - 1.0))
            yc = jnp.clip(yf, 0.0, h - 1.0).astype(jnp.int32)
            xc = jnp.clip(xf, 0.0, w - 1.0).astype(jnp.int32)
            idxv[pl.ds(k * ppw + g * L, L)] = yc * w + xc
            wgtv[pl.ds(k * ppw + g * L, L)] = jnp.where(valid, wgt, 0.0)

    pltpu.async_copy(d2_hbm.at[idxv], rows, sem).wait()

    z = jnp.zeros((L,), jnp.float32)

    def body(p, carry):
        d0, d1, n0, n1 = carry
        ws = [plsc.load_gather(wgtv, [jnp.full((L,), k * ppw + p, jnp.int32)])
              for k in range(4)]
        dv = z
        nv = z
        for ch in range(c // L):
            s = ch * L
            v = (ws[0] * rows[0 * ppw + p, pl.ds(s, L)]
                 + ws[1] * rows[1 * ppw + p, pl.ds(s, L)]
                 + ws[2] * rows[2 * ppw + p, pl.ds(s, L)]
                 + ws[3] * rows[3 * ppw + p, pl.ds(s, L)])
            kdc = kdv[p, pl.ds(s, L)]
            dv = dv + v * kdc
            nv = nv + v * v
        dot = jnp.full((L,), jnp.sum(dv))
        n2 = jnp.full((L,), jnp.sum(nv))
        sel = lanes == jnp.full((L,), p % L, jnp.int32)
        in0 = jnp.full((L,), p < L)
        d0 = jnp.where(sel & in0, dot, d0)
        n0 = jnp.where(sel & in0, n2, n0)
        d1 = jnp.where(sel & (~in0), dot, d1)
        n1 = jnp.where(sel & (~in0), n2, n1)
        return (d0, d1, n0, n1)

    accs = plsc.parallel_loop(0, ppw, carry=(z, z, z, z))(body)

    for g in range(ngroups):
        dacc = accs[g]
        nacc = accs[2 + g]
        n2 = jnp.maximum(nacc, 1e-24)
        i = plsc.bitcast(n2, jnp.int32)
        y = plsc.bitcast(0x5F3759DF - lax.shift_right_logical(i, 1),
                         jnp.float32)
        for _ in range(3):
            y = y * (1.5 - 0.5 * n2 * y * y)
        posv[pl.ds(g * L, L)] = 2.0 - 2.0 * dacc * y

    pltpu.sync_copy(posv, pos_hbm.at[0, pl.ds(base, ppw)])


def _tc_body(w_ref, kd_ref, d2t_ref, neg_ref,
             *, n_rows, m, w):
    wt = jnp.transpose(w_ref[...], (1, 0))
    px = wt[1:2, :]
    py = wt[0:1, :]
    kdt = kd_ref[...]
    d2t = d2t_ref[...]

    dmat = jax.lax.dot_general(kdt, d2t, (((0,), (0,)), ((), ())),
                               preferred_element_type=jnp.float32,
                               precision=jax.lax.Precision.HIGHEST)

    lane = jax.lax.broadcasted_iota(jnp.int32, (n_rows, m), 1)

    half = GRID_SIZE // 2

    def three_nearest(p):
        il = jnp.clip(jnp.floor((p - half) / GRID_SIZE), 0.0, w - 2.0)
        c_l = il * GRID_SIZE + half
        c_h = c_l + GRID_SIZE
        near_l = jnp.abs(p - c_l) <= jnp.abs(p - c_h)
        a0 = jnp.where(near_l, il, il + 1.0)
        a1 = jnp.where(near_l, il + 1.0, il)
        dm1 = jnp.abs(p - (c_l - GRID_SIZE))
        dp2 = jnp.abs(p - (c_h + GRID_SIZE))
        a2 = jnp.where(il == 0.0, il + 2.0,
                       jnp.where(il == w - 2.0, il - 1.0,
                                 jnp.where(dm1 <= dp2, il - 1.0, il + 2.0)))
        return a0, a1, a2

    x0c, x1c, x2c = three_nearest(px)
    y0c, y1c, y2c = three_nearest(py)
    dists, fids = [], []
    for (xc, yc) in ((x0c, y0c), (x1c, y0c), (x0c, y1c),
                     (x1c, y1c), (x2c, y0c), (x0c, y2c)):
        dx = px - (xc * GRID_SIZE + half)
        dy = py - (yc * GRID_SIZE + half)
        dists.append(jnp.sqrt(dx * dx + dy * dy))
        fids.append(yc.astype(jnp.int32) * w + xc.astype(jnp.int32))
    cnts = []
    for j in range(6):
        cnt = jnp.zeros_like(fids[j])
        for k in range(6):
            if k == j:
                continue
            less = (dists[k] < dists[j]) | ((dists[k] == dists[j])
                                            & (fids[k] < fids[j]))
            cnt = cnt + less.astype(jnp.int32)
        cnts.append(cnt)
    rank_ids = []
    for r in range(4):
        rid = jnp.zeros_like(fids[0])
        for j in range(6):
            rid = rid + jnp.where(cnts[j] == r, fids[j], 0)
        rank_ids.append(rid.reshape(n_rows, 1))
    hit = ((lane == rank_ids[0]) | (lane == rank_ids[1])
           | (lane == rank_ids[2]) | (lane == rank_ids[3]))
    dm = dmat - jnp.where(hit, 2.5, 0.0)

    excl = None
    for k in range(4):
        if excl is None:
            mx = jnp.max(dm, axis=1, keepdims=True)
        else:
            mx = jnp.max(jnp.where(excl, -jnp.inf, dm), axis=1, keepdims=True)
        neg_ref[k:k + 1, :] = 2.0 - 2.0 * mx.reshape(1, n_rows)
        if k < 3:
            hitk = dm == mx
            excl = hitk if excl is None else (excl | hitk)


def _combine_body(neg_ref, pos_ref, loss_ref, posm_ref, *, n):
    negs = neg_ref[...]
    pos = pos_ref[...]
    lsum = jnp.sum(jnp.maximum(pos - negs + MARGIN, 0.0))
    loss_ref[...] = jnp.full((1, 1), lsum * (LOSS_LAMBDA / (4.0 * n)))
    posm_ref[...] = jnp.full((1, 1), jnp.sum(pos) / n)


@jax.jit
def kernel(kp1, w_kp1, kp1_desc, desc2):
    del kp1
    n, c = kp1_desc.shape
    _, _, hc, wc = desc2.shape
    m = hc * wc
    g = n // BN
    ppw = n // NW

    d2 = jnp.transpose(desc2[0], (1, 2, 0)).reshape(m, c)

    mesh = plsc.VectorSubcoreMesh(core_axis_name="c", subcore_axis_name="s",
                                  num_cores=2, num_subcores=16)
    sc_body = functools.partial(_sc_sample_body, ppw=ppw, c=c, h=hc, w=wc)
    sc_fn = pl.kernel(
        sc_body,
        out_type=jax.ShapeDtypeStruct((1, n), jnp.float32),
        mesh=mesh,
        compiler_params=pltpu.CompilerParams(use_tc_tiling_on_sc=False,
                                             needs_layout_passes=False),
        scratch_types=[
            pltpu.VMEM((2 * ppw,), jnp.float32),
            pltpu.VMEM((4 * ppw,), jnp.int32),
            pltpu.VMEM((4 * ppw,), jnp.float32),
            pltpu.VMEM((4 * ppw, c), jnp.float32),
            pltpu.VMEM((ppw, c), jnp.float32),
            pltpu.VMEM((ppw,), jnp.float32),
            pltpu.SemaphoreType.DMA,
        ],
    )
    pos = sc_fn(w_kp1.reshape(-1), kp1_desc, d2)

    d2t = desc2[0].reshape(c, m)
    kdt = kp1_desc.T
    posl = pos

    body = functools.partial(_tc_body, n_rows=BN, m=m, w=wc)
    negs = pl.pallas_call(
        body,
        out_shape=jax.ShapeDtypeStruct((4, BN), jnp.float32),
    )(w_kp1, kdt, d2t)

    loss, posm = pl.pallas_call(
        functools.partial(_combine_body, n=n),
        out_shape=[
            jax.ShapeDtypeStruct((1, 1), jnp.float32),
            jax.ShapeDtypeStruct((1, 1), jnp.float32),
        ],
    )(negs, posl)
    return (loss[0, 0], posm[0, 0])


# speedup vs baseline: 1.0319x; 1.0319x over previous
import functools

import jax
import jax.numpy as jnp
from jax import lax
from jax.experimental import pallas as pl
from jax.experimental.pallas import tpu as pltpu
from jax.experimental.pallas import tpu_sc as plsc

GRID_SIZE = 16
MARGIN = 1.0
LOSS_LAMBDA = 1.0

BN = 1024
NW = 32
L = 16


def _sc_sample_body(w_hbm, kd_hbm, d2_hbm, pos_hbm,
                    wv, idxv, wgtv, rows, kdv, posv, sem,
                    *, ppw, c, h, w):
    wid = lax.axis_index("s") * 2 + lax.axis_index("c")
    base = wid * ppw
    pltpu.sync_copy(w_hbm.at[pl.ds(base * 2, ppw * 2)], wv)
    pltpu.sync_copy(kd_hbm.at[pl.ds(base, ppw)], kdv)

    lanes = lax.iota(jnp.int32, L)
    ngroups = ppw // L

    def floorf(v):
        t = v.astype(jnp.int32).astype(jnp.float32)
        return jnp.where(v < t, t - 1.0, t)

    for g in range(ngroups):
        pyg = plsc.load_gather(wv, [(g * L + lanes) * 2])
        pxg = plsc.load_gather(wv, [(g * L + lanes) * 2 + 1])
        ys = pyg / GRID_SIZE - 0.5
        xs = pxg / GRID_SIZE - 0.5
        y0 = floorf(ys)
        x0 = floorf(xs)
        y1 = y0 + 1.0
        x1 = x0 + 1.0
        wx1 = xs - x0
        wx0 = 1.0 - wx1
        wy1 = ys - y0
        wy0 = 1.0 - wy1
        for k, (yf, xf, wgt) in enumerate((
                (y0, x0, wy0 * wx0), (y0, x1, wy0 * wx1),
                (y1, x0, wy1 * wx0), (y1, x1, wy1 * wx1))):
            valid = ((yf >= 0.0) & (yf <= h - 1.0)
                     & (xf >= 0.0) & (xf <= w - 1.0))
            yc = jnp.clip(yf, 0.0, h - 1.0).astype(jnp.int32)
            xc = jnp.clip(xf, 0.0, w - 1.0).astype(jnp.int32)
            idxv[pl.ds(k * ppw + g * L, L)] = yc * w + xc
            wgtv[pl.ds(k * ppw + g * L, L)] = jnp.where(valid, wgt, 0.0)

    pltpu.async_copy(d2_hbm.at[idxv], rows, sem).wait()

    z = jnp.zeros((L,), jnp.float32)

    def body(p, carry):
        d0, d1, n0, n1 = carry
        ws = [plsc.load_gather(wgtv, [jnp.full((L,), k * ppw + p, jnp.int32)])
              for k in range(4)]
        dv = z
        nv = z
        for ch in range(c // L):
            s = ch * L
            v = (ws[0] * rows[0 * ppw + p, pl.ds(s, L)]
                 + ws[1] * rows[1 * ppw + p, pl.ds(s, L)]
                 + ws[2] * rows[2 * ppw + p, pl.ds(s, L)]
                 + ws[3] * rows[3 * ppw + p, pl.ds(s, L)])
            kdc = kdv[p, pl.ds(s, L)]
            dv = dv + v * kdc
            nv = nv + v * v
        dot = jnp.full((L,), jnp.sum(dv))
        n2 = jnp.full((L,), jnp.sum(nv))
        sel = lanes == jnp.full((L,), p % L, jnp.int32)
        in0 = jnp.full((L,), p < L)
        d0 = jnp.where(sel & in0, dot, d0)
        n0 = jnp.where(sel & in0, n2, n0)
        d1 = jnp.where(sel & (~in0), dot, d1)
        n1 = jnp.where(sel & (~in0), n2, n1)
        return (d0, d1, n0, n1)

    accs = plsc.parallel_loop(0, ppw, carry=(z, z, z, z))(body)

    for g in range(ngroups):
        dacc = accs[g]
        nacc = accs[2 + g]
        n2 = jnp.maximum(nacc, 1e-24)
        i = plsc.bitcast(n2, jnp.int32)
        y = plsc.bitcast(0x5F3759DF - lax.shift_right_logical(i, 1),
                         jnp.float32)
        for _ in range(3):
            y = y * (1.5 - 0.5 * n2 * y * y)
        posv[pl.ds(g * L, L)] = 2.0 - 2.0 * dacc * y

    pltpu.sync_copy(posv, pos_hbm.at[0, pl.ds(base, ppw)])


def _tc_body(wt_ref, kd_ref, d2t_ref, neg_ref,
             *, n_rows, m, w):
    px = wt_ref[1:2, :]
    py = wt_ref[0:1, :]
    kdt = kd_ref[...]
    d2t = d2t_ref[...]

    dmat = jax.lax.dot_general(kdt, d2t, (((0,), (0,)), ((), ())),
                               preferred_element_type=jnp.float32,
                               precision=jax.lax.Precision.HIGHEST)

    lane = jax.lax.broadcasted_iota(jnp.int32, (n_rows, m), 1)

    half = GRID_SIZE // 2

    def three_nearest(p):
        il = jnp.clip(jnp.floor((p - half) / GRID_SIZE), 0.0, w - 2.0)
        c_l = il * GRID_SIZE + half
        c_h = c_l + GRID_SIZE
        near_l = jnp.abs(p - c_l) <= jnp.abs(p - c_h)
        a0 = jnp.where(near_l, il, il + 1.0)
        a1 = jnp.where(near_l, il + 1.0, il)
        dm1 = jnp.abs(p - (c_l - GRID_SIZE))
        dp2 = jnp.abs(p - (c_h + GRID_SIZE))
        a2 = jnp.where(il == 0.0, il + 2.0,
                       jnp.where(il == w - 2.0, il - 1.0,
                                 jnp.where(dm1 <= dp2, il - 1.0, il + 2.0)))
        return a0, a1, a2

    x0c, x1c, x2c = three_nearest(px)
    y0c, y1c, y2c = three_nearest(py)
    dists, fids = [], []
    for (xc, yc) in ((x0c, y0c), (x1c, y0c), (x0c, y1c),
                     (x1c, y1c), (x2c, y0c), (x0c, y2c)):
        dx = px - (xc * GRID_SIZE + half)
        dy = py - (yc * GRID_SIZE + half)
        dists.append(jnp.sqrt(dx * dx + dy * dy))
        fids.append(yc.astype(jnp.int32) * w + xc.astype(jnp.int32))
    cnts = []
    for j in range(6):
        cnt = jnp.zeros_like(fids[j])
        for k in range(6):
            if k == j:
                continue
            less = (dists[k] < dists[j]) | ((dists[k] == dists[j])
                                            & (fids[k] < fids[j]))
            cnt = cnt + less.astype(jnp.int32)
        cnts.append(cnt)
    rank_ids = []
    for r in range(4):
        rid = jnp.zeros_like(fids[0])
        for j in range(6):
            rid = rid + jnp.where(cnts[j] == r, fids[j], 0)
        rank_ids.append(rid.reshape(n_rows, 1))
    hit = ((lane == rank_ids[0]) | (lane == rank_ids[1])
           | (lane == rank_ids[2]) | (lane == rank_ids[3]))
    dm = dmat - jnp.where(hit, 2.5, 0.0)

    excl = None
    for k in range(4):
        if excl is None:
            mx = jnp.max(dm, axis=1, keepdims=True)
        else:
            mx = jnp.max(jnp.where(excl, -jnp.inf, dm), axis=1, keepdims=True)
        neg_ref[k:k + 1, :] = 2.0 - 2.0 * mx.reshape(1, n_rows)
        if k < 3:
            hitk = dm == mx
            excl = hitk if excl is None else (excl | hitk)


def _combine_body(neg_ref, pos_ref, loss_ref, posm_ref, *, n):
    negs = neg_ref[...]
    pos = pos_ref[...]
    lsum = jnp.sum(jnp.maximum(pos - negs + MARGIN, 0.0))
    loss_ref[...] = jnp.full((1, 1), lsum * (LOSS_LAMBDA / (4.0 * n)))
    posm_ref[...] = jnp.full((1, 1), jnp.sum(pos) / n)


@jax.jit
def kernel(kp1, w_kp1, kp1_desc, desc2):
    del kp1
    n, c = kp1_desc.shape
    _, _, hc, wc = desc2.shape
    m = hc * wc
    g = n // BN
    ppw = n // NW

    d2 = jnp.transpose(desc2[0], (1, 2, 0)).reshape(m, c)

    mesh = plsc.VectorSubcoreMesh(core_axis_name="c", subcore_axis_name="s",
                                  num_cores=2, num_subcores=16)
    sc_body = functools.partial(_sc_sample_body, ppw=ppw, c=c, h=hc, w=wc)
    sc_fn = pl.kernel(
        sc_body,
        out_type=jax.ShapeDtypeStruct((1, n), jnp.float32),
        mesh=mesh,
        compiler_params=pltpu.CompilerParams(use_tc_tiling_on_sc=False,
                                             needs_layout_passes=False),
        scratch_types=[
            pltpu.VMEM((2 * ppw,), jnp.float32),
            pltpu.VMEM((4 * ppw,), jnp.int32),
            pltpu.VMEM((4 * ppw,), jnp.float32),
            pltpu.VMEM((4 * ppw, c), jnp.float32),
            pltpu.VMEM((ppw, c), jnp.float32),
            pltpu.VMEM((ppw,), jnp.float32),
            pltpu.SemaphoreType.DMA,
        ],
    )
    pos = sc_fn(w_kp1.reshape(-1), kp1_desc, d2)

    d2t = desc2[0].reshape(c, m)
    kdt = kp1_desc.T
    wt = w_kp1.T
    posl = pos

    body = functools.partial(_tc_body, n_rows=BN, m=m, w=wc)
    negs = pl.pallas_call(
        body,
        out_shape=jax.ShapeDtypeStruct((4, BN), jnp.float32),
    )(wt, kdt, d2t)

    loss, posm = pl.pallas_call(
        functools.partial(_combine_body, n=n),
        out_shape=[
            jax.ShapeDtypeStruct((1, 1), jnp.float32),
            jax.ShapeDtypeStruct((1, 1), jnp.float32),
        ],
    )(negs, posl)
    return (loss[0, 0], posm[0, 0])


# speedup vs baseline: 1.0431x; 1.0108x over previous
import functools

import jax
import jax.numpy as jnp
from jax import lax
from jax.experimental import pallas as pl
from jax.experimental.pallas import tpu as pltpu
from jax.experimental.pallas import tpu_sc as plsc

GRID_SIZE = 16
MARGIN = 1.0
LOSS_LAMBDA = 1.0

BN = 1024
NW = 32
L = 16


def _sc_sample_body(w_hbm, kd_hbm, d2_hbm, pos_hbm,
                    wv, idxv, wgtv, rows, kdv, posv, sem,
                    *, ppw, c, h, w):
    wid = lax.axis_index("s") * 2 + lax.axis_index("c")
    base = wid * ppw
    pltpu.sync_copy(w_hbm.at[pl.ds(base * 2, ppw * 2)], wv)
    pltpu.sync_copy(kd_hbm.at[pl.ds(base, ppw)], kdv)

    lanes = lax.iota(jnp.int32, L)
    ngroups = ppw // L

    def floorf(v):
        t = v.astype(jnp.int32).astype(jnp.float32)
        return jnp.where(v < t, t - 1.0, t)

    for g in range(ngroups):
        pyg = plsc.load_gather(wv, [(g * L + lanes) * 2])
        pxg = plsc.load_gather(wv, [(g * L + lanes) * 2 + 1])
        ys = pyg / GRID_SIZE - 0.5
        xs = pxg / GRID_SIZE - 0.5
        y0 = floorf(ys)
        x0 = floorf(xs)
        y1 = y0 + 1.0
        x1 = x0 + 1.0
        wx1 = xs - x0
        wx0 = 1.0 - wx1
        wy1 = ys - y0
        wy0 = 1.0 - wy1
        for k, (yf, xf, wgt) in enumerate((
                (y0, x0, wy0 * wx0), (y0, x1, wy0 * wx1),
                (y1, x0, wy1 * wx0), (y1, x1, wy1 * wx1))):
            valid = ((yf >= 0.0) & (yf <= h - 1.0)
                     & (xf >= 0.0) & (xf <= w - 1.0))
            yc = jnp.clip(yf, 0.0, h - 1.0).astype(jnp.int32)
            xc = jnp.clip(xf, 0.0, w - 1.0).astype(jnp.int32)
            idxv[pl.ds(k * ppw + g * L, L)] = yc * w + xc
            wgtv[pl.ds(k * ppw + g * L, L)] = jnp.where(valid, wgt, 0.0)

    pltpu.async_copy(d2_hbm.at[idxv], rows, sem).wait()

    z = jnp.zeros((L,), jnp.float32)

    def body(p, carry):
        d0, d1, n0, n1 = carry
        ws = [plsc.load_gather(wgtv, [jnp.full((L,), k * ppw + p, jnp.int32)])
              for k in range(4)]
        dv = z
        nv = z
        for ch in range(c // L):
            s = ch * L
            v = (ws[0] * rows[0 * ppw + p, pl.ds(s, L)]
                 + ws[1] * rows[1 * ppw + p, pl.ds(s, L)]
                 + ws[2] * rows[2 * ppw + p, pl.ds(s, L)]
                 + ws[3] * rows[3 * ppw + p, pl.ds(s, L)])
            kdc = kdv[p, pl.ds(s, L)]
            dv = dv + v * kdc
            nv = nv + v * v
        dot = jnp.full((L,), jnp.sum(dv))
        n2 = jnp.full((L,), jnp.sum(nv))
        sel = lanes == jnp.full((L,), p % L, jnp.int32)
        in0 = jnp.full((L,), p < L)
        d0 = jnp.where(sel & in0, dot, d0)
        n0 = jnp.where(sel & in0, n2, n0)
        d1 = jnp.where(sel & (~in0), dot, d1)
        n1 = jnp.where(sel & (~in0), n2, n1)
        return (d0, d1, n0, n1)

    accs = plsc.parallel_loop(0, ppw, carry=(z, z, z, z))(body)

    for g in range(ngroups):
        dacc = accs[g]
        nacc = accs[2 + g]
        n2 = jnp.maximum(nacc, 1e-24)
        i = plsc.bitcast(n2, jnp.int32)
        y = plsc.bitcast(0x5F3759DF - lax.shift_right_logical(i, 1),
                         jnp.float32)
        for _ in range(3):
            y = y * (1.5 - 0.5 * n2 * y * y)
        posv[pl.ds(g * L, L)] = 2.0 - 2.0 * dacc * y

    pltpu.sync_copy(posv, pos_hbm.at[0, pl.ds(base, ppw)])


def _tc_body(wt_ref, kd_ref, d2t_ref, neg_ref,
             *, n_rows, m, w):
    px = wt_ref[1:2, :]
    py = wt_ref[0:1, :]
    kdt = kd_ref[...]
    d2t = d2t_ref[...]

    dmat = jax.lax.dot_general(kdt, d2t, (((0,), (0,)), ((), ())),
                               preferred_element_type=jnp.float32,
                               precision=jax.lax.Precision.HIGHEST)

    lane = jax.lax.broadcasted_iota(jnp.int32, (n_rows, m), 1)

    half = GRID_SIZE // 2

    def three_nearest(p):
        il = jnp.clip(jnp.floor((p - half) / GRID_SIZE), 0.0, w - 2.0)
        c_l = il * GRID_SIZE + half
        c_h = c_l + GRID_SIZE
        near_l = jnp.abs(p - c_l) <= jnp.abs(p - c_h)
        a0 = jnp.where(near_l, il, il + 1.0)
        a1 = jnp.where(near_l, il + 1.0, il)
        dm1 = jnp.abs(p - (c_l - GRID_SIZE))
        dp2 = jnp.abs(p - (c_h + GRID_SIZE))
        a2 = jnp.where(il == 0.0, il + 2.0,
                       jnp.where(il == w - 2.0, il - 1.0,
                                 jnp.where(dm1 <= dp2, il - 1.0, il + 2.0)))
        return a0, a1, a2

    x0c, x1c, x2c = three_nearest(px)
    y0c, y1c, y2c = three_nearest(py)
    dists, fids = [], []
    for (xc, yc) in ((x0c, y0c), (x1c, y0c), (x0c, y1c),
                     (x1c, y1c), (x2c, y0c), (x0c, y2c)):
        dx = px - (xc * GRID_SIZE + half)
        dy = py - (yc * GRID_SIZE + half)
        dists.append(jnp.sqrt(dx * dx + dy * dy))
        fids.append(yc.astype(jnp.int32) * w + xc.astype(jnp.int32))
    cnts = []
    for j in range(6):
        cnt = jnp.zeros_like(fids[j])
        for k in range(6):
            if k == j:
                continue
            less = (dists[k] < dists[j]) | ((dists[k] == dists[j])
                                            & (fids[k] < fids[j]))
            cnt = cnt + less.astype(jnp.int32)
        cnts.append(cnt)
    rank_ids = []
    for r in range(4):
        rid = jnp.zeros_like(fids[0])
        for j in range(6):
            rid = rid + jnp.where(cnts[j] == r, fids[j], 0)
        rank_ids.append(rid.reshape(n_rows, 1))
    hit = ((lane == rank_ids[0]) | (lane == rank_ids[1])
           | (lane == rank_ids[2]) | (lane == rank_ids[3]))
    dm = dmat - jnp.where(hit, 2.5, 0.0)

    mx = None
    for k in range(4):
        if mx is None:
            mx = jnp.max(dm, axis=1, keepdims=True)
        else:
            mx = jnp.max(jnp.where(dm < mx, dm, -jnp.inf),
                         axis=1, keepdims=True)
        neg_ref[k:k + 1, :] = 2.0 - 2.0 * mx.reshape(1, n_rows)


def _combine_body(neg_ref, pos_ref, loss_ref, posm_ref, *, n):
    negs = neg_ref[...]
    pos = pos_ref[...]
    lsum = jnp.sum(jnp.maximum(pos - negs + MARGIN, 0.0))
    loss_ref[...] = jnp.full((1, 1), lsum * (LOSS_LAMBDA / (4.0 * n)))
    posm_ref[...] = jnp.full((1, 1), jnp.sum(pos) / n)


@jax.jit
def kernel(kp1, w_kp1, kp1_desc, desc2):
    del kp1
    n, c = kp1_desc.shape
    _, _, hc, wc = desc2.shape
    m = hc * wc
    g = n // BN
    ppw = n // NW

    d2 = jnp.transpose(desc2[0], (1, 2, 0)).reshape(m, c)

    mesh = plsc.VectorSubcoreMesh(core_axis_name="c", subcore_axis_name="s",
                                  num_cores=2, num_subcores=16)
    sc_body = functools.partial(_sc_sample_body, ppw=ppw, c=c, h=hc, w=wc)
    sc_fn = pl.kernel(
        sc_body,
        out_type=jax.ShapeDtypeStruct((1, n), jnp.float32),
        mesh=mesh,
        compiler_params=pltpu.CompilerParams(use_tc_tiling_on_sc=False,
                                             needs_layout_passes=False),
        scratch_types=[
            pltpu.VMEM((2 * ppw,), jnp.float32),
            pltpu.VMEM((4 * ppw,), jnp.int32),
            pltpu.VMEM((4 * ppw,), jnp.float32),
            pltpu.VMEM((4 * ppw, c), jnp.float32),
            pltpu.VMEM((ppw, c), jnp.float32),
            pltpu.VMEM((ppw,), jnp.float32),
            pltpu.SemaphoreType.DMA,
        ],
    )
    pos = sc_fn(w_kp1.reshape(-1), kp1_desc, d2)

    d2t = desc2[0].reshape(c, m)
    kdt = kp1_desc.T
    wt = w_kp1.T
    posl = pos

    body = functools.partial(_tc_body, n_rows=BN, m=m, w=wc)
    negs = pl.pallas_call(
        body,
        out_shape=jax.ShapeDtypeStruct((4, BN), jnp.float32),
    )(wt, kdt, d2t)

    loss, posm = pl.pallas_call(
        functools.partial(_combine_body, n=n),
        out_shape=[
            jax.ShapeDtypeStruct((1, 1), jnp.float32),
            jax.ShapeDtypeStruct((1, 1), jnp.float32),
        ],
    )(negs, posl)
    return (loss[0, 0], posm[0, 0])
